# baseline (speedup 1.0000x reference)
import functools

import jax
import jax.numpy as jnp
import numpy as np
from jax import lax
from jax.experimental import pallas as pl
from jax.experimental.pallas import tpu as pltpu
from jax.experimental.pallas import tpu_sc as plsc

_N = 10000
_E = 320000
_D = 128
_R = 8

_NC = 2
_NS = 16
_NW = _NC * _NS
_CHUNK = 128
_CPT = 80
_EP = _NW * _CPT * _CHUNK
_SPROWS = 10240
_RPT = _SPROWS // _NS
_TRASH = _N

_BLK = 2000
_NB = _N // _BLK


def _mm_body(x_ref, w_ref, b_ref, t_ref, s_ref):
    xb = x_ref[...].astype(jnp.bfloat16)
    for r in range(_R):
        t_ref[r] = jnp.dot(xb, w_ref[r], preferred_element_type=jnp.float32)
    s_ref[...] = (jnp.dot(xb, w_ref[_R], preferred_element_type=jnp.float32)
                  + b_ref[...])


def _transform(x, w_all, b2d):
    return pl.pallas_call(
        _mm_body,
        grid=(_NB,),
        in_specs=[
            pl.BlockSpec((_BLK, _D), lambda i: (i, 0)),
            pl.BlockSpec((_R + 1, _D, _D), lambda i: (0, 0, 0)),
            pl.BlockSpec((1, _D), lambda i: (0, 0)),
        ],
        out_specs=[
            pl.BlockSpec((_R, _BLK, _D), lambda i: (0, i, 0)),
            pl.BlockSpec((_BLK, _D), lambda i: (i, 0)),
        ],
        out_shape=[
            jax.ShapeDtypeStruct((_R, _N, _D), jnp.float32),
            jax.ShapeDtypeStruct((_N, _D), jnp.float32),
        ],
    )(x, w_all, b2d)


def _mm_combine_body(a_ref, s_ref, w_ref, b_ref, t_ref, s2_ref):
    h = jnp.maximum(a_ref[0] + a_ref[1] + s_ref[...], 0.0)
    hb = h.astype(jnp.bfloat16)
    for r in range(_R):
        t_ref[r] = jnp.dot(hb, w_ref[r], preferred_element_type=jnp.float32)
    s2_ref[...] = (jnp.dot(hb, w_ref[_R], preferred_element_type=jnp.float32)
                   + b_ref[...])


def _combine_transform(agg, selfp, w_all, b2d):
    return pl.pallas_call(
        _mm_combine_body,
        grid=(_NB,),
        in_specs=[
            pl.BlockSpec((_NC, _BLK, _D), lambda i: (0, i, 0)),
            pl.BlockSpec((_BLK, _D), lambda i: (i, 0)),
            pl.BlockSpec((_R + 1, _D, _D), lambda i: (0, 0, 0)),
            pl.BlockSpec((1, _D), lambda i: (0, 0)),
        ],
        out_specs=[
            pl.BlockSpec((_R, _BLK, _D), lambda i: (0, i, 0)),
            pl.BlockSpec((_BLK, _D), lambda i: (i, 0)),
        ],
        out_shape=[
            jax.ShapeDtypeStruct((_R, _N, _D), jnp.float32),
            jax.ShapeDtypeStruct((_N, _D), jnp.float32),
        ],
    )(agg, selfp, w_all, b2d)


def _final_body(a_ref, s_ref, o_ref):
    o_ref[...] = a_ref[0] + a_ref[1] + s_ref[...]


def _final(agg, selfp):
    return pl.pallas_call(
        _final_body,
        grid=(_NB,),
        in_specs=[
            pl.BlockSpec((_NC, _BLK, _D), lambda i: (0, i, 0)),
            pl.BlockSpec((_BLK, _D), lambda i: (i, 0)),
        ],
        out_specs=pl.BlockSpec((_BLK, _D), lambda i: (i, 0)),
        out_shape=jax.ShapeDtypeStruct((_N, _D), jnp.float32),
    )(agg, selfp)


_NBUF = 2
_NPH = 2
_PC = _CPT // _NPH
_CROWS = _E // _CHUNK
_PADC = 4
_LASTC = _CROWS + _PADC - 31 * _CPT


def _edge_agg(t, rowidx, dstidx):
    mesh = plsc.VectorSubcoreMesh(core_axis_name="c", subcore_axis_name="s")

    @functools.partial(
        pl.kernel,
        mesh=mesh,
        out_type=jax.ShapeDtypeStruct((_NC * _SPROWS, _D), jnp.float32),
        scratch_types=[
            pltpu.VMEM((_PC * _CHUNK,), jnp.int32),
            pltpu.VMEM((_PC, _CHUNK), jnp.int32),
            pltpu.VMEM((_CHUNK, _D), jnp.float32),
            pltpu.VMEM((_CHUNK, _D), jnp.float32),
            pltpu.VMEM_SHARED((_SPROWS, _D), jnp.float32),
            pltpu.SemaphoreType.DMA,
            pltpu.SemaphoreType.DMA,
        ],
    )
    def k(t_hbm, ri_hbm, di_hbm, out_hbm, idx_all, dst_all,
          b0, b1, agg_sh, s0, s1):
        bufs = (b0, b1)
        sems = (s0, s1)
        c = lax.axis_index("c")
        s = lax.axis_index("s")
        wid = s * _NC + c
        last = wid == _NW - 1

        @pl.loop(0, _CHUNK)
        def _(i):
            @pl.loop(0, _D // 16)
            def _(j):
                b0[i, pl.ds(j * 16, 16)] = jnp.zeros((16,), jnp.float32)

        @pl.loop(0, _RPT // _CHUNK)
        def _(kk):
            pltpu.sync_copy(b0, agg_sh.at[pl.ds(s * _RPT + kk * _CHUNK,
                                                _CHUNK)])

        plsc.subcore_barrier()

        for p in range(_NPH):
            pcnt = jnp.where(last, _LASTC if p == 0 else 0, _PC)
            row0 = wid * _CPT + p * _PC

            @pl.when(jnp.logical_not(last))
            def _():
                pltpu.sync_copy(ri_hbm.at[pl.ds(row0 * _CHUNK,
                                                _PC * _CHUNK)], idx_all)
                pltpu.sync_copy(di_hbm.at[pl.ds(row0, _PC)], dst_all)

            if p == 0:
                @pl.when(last)
                def _():
                    pltpu.sync_copy(
                        ri_hbm.at[pl.ds(row0 * _CHUNK, _LASTC * _CHUNK)],
                        idx_all.at[pl.ds(0, _LASTC * _CHUNK)])
                    pltpu.sync_copy(di_hbm.at[pl.ds(row0, _LASTC)],
                                    dst_all.at[pl.ds(0, _LASTC)])

            @pl.when(pcnt > 0)
            def _():
                for b in range(_NBUF):
                    pltpu.async_copy(
                        t_hbm.at[idx_all.at[pl.ds(b * _CHUNK, _CHUNK)]],
                        bufs[b], sems[b])

            @pl.loop(0, pcnt // _NBUF)
            def _(kk):
                base = kk * _NBUF
                for b in range(_NBUF):
                    pltpu.make_async_copy(t_hbm.at[pl.ds(0, _CHUNK)],
                                          bufs[b], sems[b]).wait()
                    pltpu.sync_copy(bufs[b], agg_sh.at[dst_all.at[base + b]],
                                    add=True)
                    nxt = base + _NBUF + b

                    @pl.when(nxt < pcnt)
                    def _():
                        pltpu.async_copy(
                            t_hbm.at[idx_all.at[pl.ds(nxt * _CHUNK,
                                                      _CHUNK)]],
                            bufs[b], sems[b])

        plsc.subcore_barrier()

        for kk in range(_RPT // _CHUNK):
            row0 = s * _RPT + kk * _CHUNK
            pltpu.async_copy(agg_sh.at[pl.ds(row0, _CHUNK)],
                             out_hbm.at[pl.ds(c * _SPROWS + row0, _CHUNK)],
                             s0)
        for kk in range(_RPT // _CHUNK):
            pltpu.make_async_copy(agg_sh.at[pl.ds(0, _CHUNK)],
                                  out_hbm.at[pl.ds(0, _CHUNK)], s0).wait()

    return k(t, rowidx, dstidx)


def kernel(feats, edge_index, etypes, W1, loop1, b1, W2, loop2, b2):
    pad_iota = np.arange(_PADC * _CHUNK, dtype=np.int32)
    rowidx_p = jnp.concatenate(
        [etypes * _N + edge_index[0], jnp.asarray(pad_iota % (_R * _N))])
    dst_p = jnp.concatenate(
        [edge_index[1].reshape(_CROWS, _CHUNK),
         jnp.asarray(_TRASH + pad_iota % (_SPROWS - _N)).reshape(_PADC,
                                                                 _CHUNK)])

    w_all1 = jnp.concatenate([W1, loop1[None]], axis=0).astype(jnp.bfloat16)
    w_all2 = jnp.concatenate([W2, loop2[None]], axis=0).astype(jnp.bfloat16)
    b1_2d = b1.reshape(1, _D)
    b2_2d = b2.reshape(1, _D)

    t1, s1 = _transform(feats, w_all1, b1_2d)
    agg1 = _edge_agg(t1.reshape(_R * _N, _D), rowidx_p, dst_p)
    t2, s2 = _combine_transform(agg1.reshape(_NC, _SPROWS, _D), s1,
                                w_all2, b2_2d)
    agg2 = _edge_agg(t2.reshape(_R * _N, _D), rowidx_p, dst_p)
    return _final(agg2.reshape(_NC, _SPROWS, _D), s2)


# speedup vs baseline: 1.0025x; 1.0025x over previous
import functools

import jax
import jax.numpy as jnp
from jax import lax
from jax.experimental import pallas as pl
from jax.experimental.pallas import tpu as pltpu
from jax.experimental.pallas import tpu_sc as plsc

_N = 10000
_E = 320000
_D = 128
_R = 8

_NC = 2
_NS = 16
_NW = _NC * _NS
_CHUNK = 128
_CPT = 80
_EP = _NW * _CPT * _CHUNK
_SPROWS = 10240
_RPT = _SPROWS // _NS
_TRASH = _N

_BLK = 2000
_NB = _N // _BLK


def _mm_body(x_ref, w_ref, b_ref, t_ref, s_ref):
    xb = x_ref[...].astype(jnp.bfloat16)
    for r in range(_R):
        t_ref[r] = jnp.dot(xb, w_ref[r], preferred_element_type=jnp.float32)
    s_ref[...] = (jnp.dot(xb, w_ref[_R], preferred_element_type=jnp.float32)
                  + b_ref[...])


def _transform(x, w_all, b2d):
    return pl.pallas_call(
        _mm_body,
        grid=(_NB,),
        in_specs=[
            pl.BlockSpec((_BLK, _D), lambda i: (i, 0)),
            pl.BlockSpec((_R + 1, _D, _D), lambda i: (0, 0, 0)),
            pl.BlockSpec((1, _D), lambda i: (0, 0)),
        ],
        out_specs=[
            pl.BlockSpec((_R, _BLK, _D), lambda i: (0, i, 0)),
            pl.BlockSpec((_BLK, _D), lambda i: (i, 0)),
        ],
        out_shape=[
            jax.ShapeDtypeStruct((_R, _N, _D), jnp.float32),
            jax.ShapeDtypeStruct((_N, _D), jnp.float32),
        ],
    )(x, w_all, b2d)


def _mm_combine_body(a_ref, s_ref, w_ref, b_ref, t_ref, s2_ref):
    h = jnp.maximum(a_ref[0] + a_ref[1] + s_ref[...], 0.0)
    hb = h.astype(jnp.bfloat16)
    for r in range(_R):
        t_ref[r] = jnp.dot(hb, w_ref[r], preferred_element_type=jnp.float32)
    s2_ref[...] = (jnp.dot(hb, w_ref[_R], preferred_element_type=jnp.float32)
                   + b_ref[...])


def _combine_transform(agg, selfp, w_all, b2d):
    return pl.pallas_call(
        _mm_combine_body,
        grid=(_NB,),
        in_specs=[
            pl.BlockSpec((_NC, _BLK, _D), lambda i: (0, i, 0)),
            pl.BlockSpec((_BLK, _D), lambda i: (i, 0)),
            pl.BlockSpec((_R + 1, _D, _D), lambda i: (0, 0, 0)),
            pl.BlockSpec((1, _D), lambda i: (0, 0)),
        ],
        out_specs=[
            pl.BlockSpec((_R, _BLK, _D), lambda i: (0, i, 0)),
            pl.BlockSpec((_BLK, _D), lambda i: (i, 0)),
        ],
        out_shape=[
            jax.ShapeDtypeStruct((_R, _N, _D), jnp.float32),
            jax.ShapeDtypeStruct((_N, _D), jnp.float32),
        ],
    )(agg, selfp, w_all, b2d)


def _final_body(a_ref, s_ref, o_ref):
    o_ref[...] = a_ref[0] + a_ref[1] + s_ref[...]


def _final(agg, selfp):
    return pl.pallas_call(
        _final_body,
        grid=(_NB,),
        in_specs=[
            pl.BlockSpec((_NC, _BLK, _D), lambda i: (0, i, 0)),
            pl.BlockSpec((_BLK, _D), lambda i: (i, 0)),
        ],
        out_specs=pl.BlockSpec((_BLK, _D), lambda i: (i, 0)),
        out_shape=jax.ShapeDtypeStruct((_N, _D), jnp.float32),
    )(agg, selfp)


_EROWS = _E // _CHUNK
_PROWS = _EP // _CHUNK


def _prep_body(ei_ref, et_ref, ri_ref, di_ref):
    i = pl.program_id(0)
    grow = i * _CHUNK + lax.broadcasted_iota(jnp.int32, (_CHUNK, _CHUNK), 0)
    col = lax.broadcasted_iota(jnp.int32, (_CHUNK, _CHUNK), 1)
    flat = grow * _CHUNK + col
    p = flat - _E
    real = flat < _E
    ri_ref[...] = jnp.where(real, et_ref[...] * _N + ei_ref[0], p)
    di_ref[...] = jnp.where(
        real, ei_ref[1],
        _TRASH + lax.rem(p, jnp.int32(_SPROWS - _N)))


def _prep(edge_index3d, etypes2d):
    return pl.pallas_call(
        _prep_body,
        grid=(_PROWS // _CHUNK,),
        in_specs=[
            pl.BlockSpec((2, _CHUNK, _CHUNK), lambda i: (0, i, 0)),
            pl.BlockSpec((_CHUNK, _CHUNK), lambda i: (i, 0)),
        ],
        out_specs=[
            pl.BlockSpec((_CHUNK, _CHUNK), lambda i: (i, 0)),
            pl.BlockSpec((_CHUNK, _CHUNK), lambda i: (i, 0)),
        ],
        out_shape=[
            jax.ShapeDtypeStruct((_PROWS, _CHUNK), jnp.int32),
            jax.ShapeDtypeStruct((_PROWS, _CHUNK), jnp.int32),
        ],
    )(edge_index3d, etypes2d)


_NBUF = 2
_NPH = 2
_PC = _CPT // _NPH


def _edge_agg(t, rowidx, dstidx):
    mesh = plsc.VectorSubcoreMesh(core_axis_name="c", subcore_axis_name="s")

    @functools.partial(
        pl.kernel,
        mesh=mesh,
        out_type=jax.ShapeDtypeStruct((_NC * _SPROWS, _D), jnp.float32),
        scratch_types=[
            pltpu.VMEM((_PC, _CHUNK), jnp.int32),
            pltpu.VMEM((_PC, _CHUNK), jnp.int32),
            pltpu.VMEM((_CHUNK, _D), jnp.float32),
            pltpu.VMEM((_CHUNK, _D), jnp.float32),
            pltpu.VMEM_SHARED((_SPROWS, _D), jnp.float32),
            pltpu.SemaphoreType.DMA,
            pltpu.SemaphoreType.DMA,
        ],
    )
    def k(t_hbm, ri_hbm, di_hbm, out_hbm, idx_all, dst_all,
          b0, b1, agg_sh, s0, s1):
        bufs = (b0, b1)
        sems = (s0, s1)
        c = lax.axis_index("c")
        s = lax.axis_index("s")
        wid = s * _NC + c

        @pl.loop(0, _CHUNK)
        def _(i):
            @pl.loop(0, _D // 16)
            def _(j):
                b0[i, pl.ds(j * 16, 16)] = jnp.zeros((16,), jnp.float32)

        @pl.loop(0, _RPT // _CHUNK)
        def _(kk):
            pltpu.sync_copy(b0, agg_sh.at[pl.ds(s * _RPT + kk * _CHUNK,
                                                _CHUNK)])

        plsc.subcore_barrier()

        for p in range(_NPH):
            pltpu.sync_copy(ri_hbm.at[pl.ds(wid * _CPT + p * _PC, _PC)],
                            idx_all)
            pltpu.sync_copy(di_hbm.at[pl.ds(wid * _CPT + p * _PC, _PC)],
                            dst_all)
            for b in range(_NBUF):
                pltpu.async_copy(t_hbm.at[idx_all.at[b]], bufs[b], sems[b])

            @pl.loop(0, _PC // _NBUF)
            def _(kk):
                base = kk * _NBUF
                for b in range(_NBUF):
                    pltpu.make_async_copy(t_hbm.at[pl.ds(0, _CHUNK)],
                                          bufs[b], sems[b]).wait()
                    pltpu.sync_copy(bufs[b], agg_sh.at[dst_all.at[base + b]],
                                    add=True)
                    nxt = base + _NBUF + b

                    @pl.when(nxt < _PC)
                    def _():
                        pltpu.async_copy(t_hbm.at[idx_all.at[nxt]],
                                         bufs[b], sems[b])

        plsc.subcore_barrier()

        for kk in range(_RPT // _CHUNK):
            row0 = s * _RPT + kk * _CHUNK
            pltpu.async_copy(agg_sh.at[pl.ds(row0, _CHUNK)],
                             out_hbm.at[pl.ds(c * _SPROWS + row0, _CHUNK)],
                             s0)
        for kk in range(_RPT // _CHUNK):
            pltpu.make_async_copy(agg_sh.at[pl.ds(0, _CHUNK)],
                                  out_hbm.at[pl.ds(0, _CHUNK)], s0).wait()

    return k(t, rowidx, dstidx)


def kernel(feats, edge_index, etypes, W1, loop1, b1, W2, loop2, b2):
    rowidx_p, dst_p = _prep(edge_index.reshape(2, _EROWS, _CHUNK),
                            etypes.reshape(_EROWS, _CHUNK))

    w_all1 = jnp.concatenate([W1, loop1[None]], axis=0).astype(jnp.bfloat16)
    w_all2 = jnp.concatenate([W2, loop2[None]], axis=0).astype(jnp.bfloat16)
    b1_2d = b1.reshape(1, _D)
    b2_2d = b2.reshape(1, _D)

    t1, s1 = _transform(feats, w_all1, b1_2d)
    agg1 = _edge_agg(t1.reshape(_R * _N, _D), rowidx_p, dst_p)
    t2, s2 = _combine_transform(agg1.reshape(_NC, _SPROWS, _D), s1,
                                w_all2, b2_2d)
    agg2 = _edge_agg(t2.reshape(_R * _N, _D), rowidx_p, dst_p)
    return _final(agg2.reshape(_NC, _SPROWS, _D), s2)
